# Initial kernel scaffold; baseline (speedup 1.0000x reference)
import jax
import jax.numpy as jnp
from jax.experimental import pallas as pl


def kernel(xyz1, xyz2, points1, points2, W0, b0, g0, beta0, W1, b1, g1, beta1):
    raise NotImplementedError("write your pallas kernel here")


# speedup vs baseline: 38.2374x; 38.2374x over previous
import functools

import jax
import jax.numpy as jnp
from jax.experimental import pallas as pl
from jax.experimental.pallas import tpu as pltpu

B, N, S, D = 8, 4096, 1024, 128
H0, H1 = 256, 128
TN = 512
NT = N // TN
INV_BN = 1.0 / (1.0 + 1e-05) ** 0.5


def _body(x1_ref, x2t_ref, p1_ref, p2_ref, w0_ref, b0_ref, g0_ref,
          beta0_ref, w1_ref, b1_ref, g1_ref, beta1_ref, out_ref, acc_ref):
    t = pl.program_id(1)

    x1 = x1_ref[0]
    x2t = x2t_ref[0]

    cross = jnp.dot(x2t, x1, preferred_element_type=jnp.float32)
    x1sq = jnp.sum(x1 * x1, axis=0, keepdims=True)
    x2sq = jnp.sum(x2t * x2t, axis=1, keepdims=True)
    dist = (x2sq - 2.0 * cross) + x1sq

    iota = jax.lax.broadcasted_iota(jnp.int32, (S, TN), 0)

    d = dist
    wmat = jnp.zeros((S, TN), jnp.float32)
    total = jnp.zeros((1, TN), jnp.float32)
    for _ in range(3):
        m = jnp.min(d, axis=0, keepdims=True)
        idx = jnp.min(jnp.where(d == m, iota, S), axis=0, keepdims=True)
        hot = iota == idx
        r = 1.0 / (m + 1e-08)
        wmat = wmat + hot.astype(jnp.float32) * r
        total = total + r
        d = jnp.where(hot, jnp.inf, d)

    interp = jnp.dot(p2_ref[0], wmat, preferred_element_type=jnp.float32)
    interp = interp * (1.0 / total)

    w0 = w0_ref[...]
    contrib = jnp.concatenate(
        [jnp.dot(p1_ref[0], w0, preferred_element_type=jnp.float32),
         jnp.dot(interp, w0, preferred_element_type=jnp.float32)], axis=0)

    @pl.when(t == 0)
    def _():
        acc_ref[...] = jnp.zeros_like(acc_ref)

    acc_ref[...] += contrib

    @pl.when(t == NT - 1)
    def _():
        h = acc_ref[...] + b0_ref[...]
        h = jnp.maximum(h * INV_BN * g0_ref[...] + beta0_ref[...], 0.0)
        h = jnp.dot(h, w1_ref[...], preferred_element_type=jnp.float32)
        h = h + b1_ref[...]
        h = jnp.maximum(h * INV_BN * g1_ref[...] + beta1_ref[...], 0.0)
        out_ref[0] = h


@jax.jit
def kernel(xyz1, xyz2, points1, points2, W0, b0, g0, beta0, W1, b1, g1, beta1):
    xyz2t = jnp.transpose(xyz2, (0, 2, 1))

    grid = (B, NT)
    out = pl.pallas_call(
        _body,
        grid=grid,
        in_specs=[
            pl.BlockSpec((1, 3, TN), lambda b, t: (b, 0, t)),
            pl.BlockSpec((1, S, 3), lambda b, t: (b, 0, 0)),
            pl.BlockSpec((1, D, TN), lambda b, t: (b, 0, t)),
            pl.BlockSpec((1, D, S), lambda b, t: (b, 0, 0)),
            pl.BlockSpec((TN, H0), lambda b, t: (t, 0)),
            pl.BlockSpec((1, H0), lambda b, t: (0, 0)),
            pl.BlockSpec((1, H0), lambda b, t: (0, 0)),
            pl.BlockSpec((1, H0), lambda b, t: (0, 0)),
            pl.BlockSpec((H0, H1), lambda b, t: (0, 0)),
            pl.BlockSpec((1, H1), lambda b, t: (0, 0)),
            pl.BlockSpec((1, H1), lambda b, t: (0, 0)),
            pl.BlockSpec((1, H1), lambda b, t: (0, 0)),
        ],
        out_specs=pl.BlockSpec((1, 2 * D, H1), lambda b, t: (b, 0, 0)),
        out_shape=jax.ShapeDtypeStruct((B, 2 * D, H1), jnp.float32),
        scratch_shapes=[pltpu.VMEM((2 * D, H0), jnp.float32)],
    )(xyz1, xyz2t, points1, points2, W0,
      b0.reshape(1, H0), g0.reshape(1, H0), beta0.reshape(1, H0),
      W1, b1.reshape(1, H1), g1.reshape(1, H1), beta1.reshape(1, H1))
    return out


# speedup vs baseline: 41.8724x; 1.0951x over previous
import functools

import jax
import jax.numpy as jnp
from jax.experimental import pallas as pl
from jax.experimental.pallas import tpu as pltpu

B, N, S, D = 8, 4096, 1024, 128
H0, H1 = 256, 128
TN = 512
NT = N // TN
INV_BN = 1.0 / (1.0 + 1e-05) ** 0.5


def _body(x1_ref, x2t_ref, p1_ref, p2_ref, w0_ref, b0_ref, g0_ref,
          beta0_ref, w1_ref, b1_ref, g1_ref, beta1_ref, out_ref, acc_ref):
    t = pl.program_id(1)

    x1 = x1_ref[0]
    x2t = x2t_ref[0]

    cross = jnp.dot(x2t, -2.0 * x1, preferred_element_type=jnp.float32)
    x1sq = jnp.sum(x1 * x1, axis=0, keepdims=True)
    x2sq = jnp.sum(x2t * x2t, axis=1, keepdims=True)
    dist = (x2sq + cross) + x1sq

    iota = jax.lax.broadcasted_iota(jnp.int32, (S, TN), 0).astype(jnp.float32)
    BIG = float(S)

    d = dist
    ms, idxs = [], []
    for k in range(3):
        m = jnp.min(d, axis=0, keepdims=True)
        idx = jnp.min(jnp.where(d == m, iota, BIG), axis=0, keepdims=True)
        ms.append(m)
        idxs.append(idx)
        if k < 2:
            d = jnp.where(iota == idx, jnp.inf, d)

    rs = [1.0 / (m + 1e-08) for m in ms]
    total = rs[0] + rs[1] + rs[2]
    wmat = (jnp.where(iota == idxs[0], rs[0], 0.0)
            + jnp.where(iota == idxs[1], rs[1], 0.0)
            + jnp.where(iota == idxs[2], rs[2], 0.0))

    interp = jnp.dot(p2_ref[0], wmat, preferred_element_type=jnp.float32)
    interp = interp * (1.0 / total)

    w0 = w0_ref[...]
    contrib = jnp.concatenate(
        [jnp.dot(p1_ref[0], w0, preferred_element_type=jnp.float32),
         jnp.dot(interp, w0, preferred_element_type=jnp.float32)], axis=0)

    @pl.when(t == 0)
    def _():
        acc_ref[...] = jnp.zeros_like(acc_ref)

    acc_ref[...] += contrib

    @pl.when(t == NT - 1)
    def _():
        h = acc_ref[...] + b0_ref[...]
        h = jnp.maximum(h * INV_BN * g0_ref[...] + beta0_ref[...], 0.0)
        h = jnp.dot(h, w1_ref[...], preferred_element_type=jnp.float32)
        h = h + b1_ref[...]
        h = jnp.maximum(h * INV_BN * g1_ref[...] + beta1_ref[...], 0.0)
        out_ref[0] = h


@jax.jit
def kernel(xyz1, xyz2, points1, points2, W0, b0, g0, beta0, W1, b1, g1, beta1):
    xyz2t = jnp.transpose(xyz2, (0, 2, 1))

    grid = (B, NT)
    out = pl.pallas_call(
        _body,
        grid=grid,
        in_specs=[
            pl.BlockSpec((1, 3, TN), lambda b, t: (b, 0, t)),
            pl.BlockSpec((1, S, 3), lambda b, t: (b, 0, 0)),
            pl.BlockSpec((1, D, TN), lambda b, t: (b, 0, t)),
            pl.BlockSpec((1, D, S), lambda b, t: (b, 0, 0)),
            pl.BlockSpec((TN, H0), lambda b, t: (t, 0)),
            pl.BlockSpec((1, H0), lambda b, t: (0, 0)),
            pl.BlockSpec((1, H0), lambda b, t: (0, 0)),
            pl.BlockSpec((1, H0), lambda b, t: (0, 0)),
            pl.BlockSpec((H0, H1), lambda b, t: (0, 0)),
            pl.BlockSpec((1, H1), lambda b, t: (0, 0)),
            pl.BlockSpec((1, H1), lambda b, t: (0, 0)),
            pl.BlockSpec((1, H1), lambda b, t: (0, 0)),
        ],
        out_specs=pl.BlockSpec((1, 2 * D, H1), lambda b, t: (b, 0, 0)),
        out_shape=jax.ShapeDtypeStruct((B, 2 * D, H1), jnp.float32),
        scratch_shapes=[pltpu.VMEM((2 * D, H0), jnp.float32)],
    )(xyz1, xyz2t, points1, points2, W0,
      b0.reshape(1, H0), g0.reshape(1, H0), beta0.reshape(1, H0),
      W1, b1.reshape(1, H1), g1.reshape(1, H1), beta1.reshape(1, H1))
    return out


# speedup vs baseline: 44.4628x; 1.0619x over previous
import functools

import jax
import jax.numpy as jnp
from jax.experimental import pallas as pl
from jax.experimental.pallas import tpu as pltpu

B, N, S, D = 8, 4096, 1024, 128
H0, H1 = 256, 128
TN = 512
NT = N // TN
INV_BN = 1.0 / (1.0 + 1e-05) ** 0.5


def _body(x1_ref, x2t_ref, p1_ref, p2_ref, w0_ref, b0_ref, g0_ref,
          beta0_ref, w1_ref, b1_ref, g1_ref, beta1_ref, out_ref, acc_ref):
    t = pl.program_id(1)

    x1 = x1_ref[0]
    x2t = x2t_ref[0]

    cross = jnp.dot(x2t, -2.0 * x1, preferred_element_type=jnp.float32)
    x1sq = jnp.sum(x1 * x1, axis=0, keepdims=True)
    x2sq = jnp.sum(x2t * x2t, axis=1, keepdims=True)
    dist = (cross + x1sq) + x2sq

    iota = jax.lax.broadcasted_iota(jnp.int32, (S, TN), 0).astype(jnp.float32)
    BIG = float(S)

    d = dist
    ms, idxs = [], []
    for k in range(3):
        m = jnp.min(d, axis=0, keepdims=True)
        idx = jnp.min(jnp.where(d == m, iota, BIG), axis=0, keepdims=True)
        ms.append(m)
        idxs.append(idx)
        if k < 2:
            d = jnp.where(iota == idx, jnp.inf, d)

    rs = [1.0 / (m + 1e-08) for m in ms]
    total = rs[0] + rs[1] + rs[2]
    wmat = (jnp.where(iota == idxs[0], rs[0], 0.0)
            + jnp.where(iota == idxs[1], rs[1], 0.0)
            + jnp.where(iota == idxs[2], rs[2], 0.0))

    interp = jnp.dot(p2_ref[0], wmat, preferred_element_type=jnp.float32)
    interp = interp * (1.0 / total)

    w0 = w0_ref[...]
    contrib = jnp.concatenate(
        [jnp.dot(p1_ref[0], w0, preferred_element_type=jnp.float32),
         jnp.dot(interp, w0, preferred_element_type=jnp.float32)], axis=0)

    @pl.when(t == 0)
    def _():
        acc_ref[...] = jnp.zeros_like(acc_ref)

    acc_ref[...] += contrib

    @pl.when(t == NT - 1)
    def _():
        h = acc_ref[...] + b0_ref[...]
        h = jnp.maximum(h * INV_BN * g0_ref[...] + beta0_ref[...], 0.0)
        h = jnp.dot(h, w1_ref[...], preferred_element_type=jnp.float32)
        h = h + b1_ref[...]
        h = jnp.maximum(h * INV_BN * g1_ref[...] + beta1_ref[...], 0.0)
        out_ref[0] = h


@jax.jit
def kernel(xyz1, xyz2, points1, points2, W0, b0, g0, beta0, W1, b1, g1, beta1):
    xyz2t = jnp.transpose(xyz2, (0, 2, 1))

    grid = (B, NT)
    out = pl.pallas_call(
        _body,
        grid=grid,
        in_specs=[
            pl.BlockSpec((1, 3, TN), lambda b, t: (b, 0, t)),
            pl.BlockSpec((1, S, 3), lambda b, t: (b, 0, 0)),
            pl.BlockSpec((1, D, TN), lambda b, t: (b, 0, t)),
            pl.BlockSpec((1, D, S), lambda b, t: (b, 0, 0)),
            pl.BlockSpec((TN, H0), lambda b, t: (t, 0)),
            pl.BlockSpec((1, H0), lambda b, t: (0, 0)),
            pl.BlockSpec((1, H0), lambda b, t: (0, 0)),
            pl.BlockSpec((1, H0), lambda b, t: (0, 0)),
            pl.BlockSpec((H0, H1), lambda b, t: (0, 0)),
            pl.BlockSpec((1, H1), lambda b, t: (0, 0)),
            pl.BlockSpec((1, H1), lambda b, t: (0, 0)),
            pl.BlockSpec((1, H1), lambda b, t: (0, 0)),
        ],
        out_specs=pl.BlockSpec((1, 2 * D, H1), lambda b, t: (b, 0, 0)),
        out_shape=jax.ShapeDtypeStruct((B, 2 * D, H1), jnp.float32),
        scratch_shapes=[pltpu.VMEM((2 * D, H0), jnp.float32)],
    )(xyz1, xyz2t, points1, points2, W0,
      b0.reshape(1, H0), g0.reshape(1, H0), beta0.reshape(1, H0),
      W1, b1.reshape(1, H1), g1.reshape(1, H1), beta1.reshape(1, H1))
    return out


# speedup vs baseline: 49.6287x; 1.1162x over previous
import functools

import jax
import jax.numpy as jnp
from jax.experimental import pallas as pl
from jax.experimental.pallas import tpu as pltpu

B, N, S, D = 8, 4096, 1024, 128
H0, H1 = 256, 128
TN = 1024
NT = N // TN
INV_BN = 1.0 / (1.0 + 1e-05) ** 0.5


def _body(x1_ref, x2t_ref, p1_ref, p2_ref, w0_ref, b0_ref, g0_ref,
          beta0_ref, w1_ref, b1_ref, g1_ref, beta1_ref, out_ref, acc_ref):
    t = pl.program_id(1)

    x1 = x1_ref[0]
    x2t = x2t_ref[0]

    cross = jnp.dot(x2t, -2.0 * x1, preferred_element_type=jnp.float32)
    x1sq = jnp.sum(x1 * x1, axis=0, keepdims=True)
    x2sq = jnp.sum(x2t * x2t, axis=1, keepdims=True)
    dist = (cross + x1sq) + x2sq

    iota = jax.lax.broadcasted_iota(jnp.int32, (S, TN), 0).astype(jnp.float32)
    BIG = float(S)

    d = dist
    ms, idxs = [], []
    for k in range(3):
        m = jnp.min(d, axis=0, keepdims=True)
        idx = jnp.min(jnp.where(d == m, iota, BIG), axis=0, keepdims=True)
        ms.append(m)
        idxs.append(idx)
        if k < 2:
            d = jnp.where(iota == idx, jnp.inf, d)

    rs = [1.0 / (m + 1e-08) for m in ms]
    total = rs[0] + rs[1] + rs[2]
    wmat = (jnp.where(iota == idxs[0], rs[0], 0.0)
            + jnp.where(iota == idxs[1], rs[1], 0.0)
            + jnp.where(iota == idxs[2], rs[2], 0.0))

    interp = jnp.dot(p2_ref[0], wmat, preferred_element_type=jnp.float32)
    interp = interp * (1.0 / total)

    w0 = w0_ref[...]
    contrib = jnp.concatenate(
        [jnp.dot(p1_ref[0], w0, preferred_element_type=jnp.float32),
         jnp.dot(interp, w0, preferred_element_type=jnp.float32)], axis=0)

    @pl.when(t == 0)
    def _():
        acc_ref[...] = jnp.zeros_like(acc_ref)

    acc_ref[...] += contrib

    @pl.when(t == NT - 1)
    def _():
        h = acc_ref[...] + b0_ref[...]
        h = jnp.maximum(h * INV_BN * g0_ref[...] + beta0_ref[...], 0.0)
        h = jnp.dot(h, w1_ref[...], preferred_element_type=jnp.float32)
        h = h + b1_ref[...]
        h = jnp.maximum(h * INV_BN * g1_ref[...] + beta1_ref[...], 0.0)
        out_ref[0] = h


@jax.jit
def kernel(xyz1, xyz2, points1, points2, W0, b0, g0, beta0, W1, b1, g1, beta1):
    xyz2t = jnp.transpose(xyz2, (0, 2, 1))

    grid = (B, NT)
    out = pl.pallas_call(
        _body,
        grid=grid,
        in_specs=[
            pl.BlockSpec((1, 3, TN), lambda b, t: (b, 0, t)),
            pl.BlockSpec((1, S, 3), lambda b, t: (b, 0, 0)),
            pl.BlockSpec((1, D, TN), lambda b, t: (b, 0, t)),
            pl.BlockSpec((1, D, S), lambda b, t: (b, 0, 0)),
            pl.BlockSpec((TN, H0), lambda b, t: (t, 0)),
            pl.BlockSpec((1, H0), lambda b, t: (0, 0)),
            pl.BlockSpec((1, H0), lambda b, t: (0, 0)),
            pl.BlockSpec((1, H0), lambda b, t: (0, 0)),
            pl.BlockSpec((H0, H1), lambda b, t: (0, 0)),
            pl.BlockSpec((1, H1), lambda b, t: (0, 0)),
            pl.BlockSpec((1, H1), lambda b, t: (0, 0)),
            pl.BlockSpec((1, H1), lambda b, t: (0, 0)),
        ],
        out_specs=pl.BlockSpec((1, 2 * D, H1), lambda b, t: (b, 0, 0)),
        out_shape=jax.ShapeDtypeStruct((B, 2 * D, H1), jnp.float32),
        scratch_shapes=[pltpu.VMEM((2 * D, H0), jnp.float32)],
    )(xyz1, xyz2t, points1, points2, W0,
      b0.reshape(1, H0), g0.reshape(1, H0), beta0.reshape(1, H0),
      W1, b1.reshape(1, H1), g1.reshape(1, H1), beta1.reshape(1, H1))
    return out


# speedup vs baseline: 50.4207x; 1.0160x over previous
import functools

import jax
import jax.numpy as jnp
from jax.experimental import pallas as pl
from jax.experimental.pallas import tpu as pltpu

B, N, S, D = 8, 4096, 1024, 128
H0, H1 = 256, 128
TN = 2048
NT = N // TN
INV_BN = 1.0 / (1.0 + 1e-05) ** 0.5


def _body(x1_ref, x2t_ref, p1_ref, p2_ref, w0_ref, b0_ref, g0_ref,
          beta0_ref, w1_ref, b1_ref, g1_ref, beta1_ref, out_ref, acc_ref):
    t = pl.program_id(1)

    x1 = x1_ref[0]
    x2t = x2t_ref[0]

    cross = jnp.dot(x2t, -2.0 * x1, preferred_element_type=jnp.float32)
    x1sq = jnp.sum(x1 * x1, axis=0, keepdims=True)
    x2sq = jnp.sum(x2t * x2t, axis=1, keepdims=True)
    dist = (cross + x1sq) + x2sq

    iota = jax.lax.broadcasted_iota(jnp.int32, (S, TN), 0).astype(jnp.float32)
    BIG = float(S)

    d = dist
    ms, idxs = [], []
    for k in range(3):
        m = jnp.min(d, axis=0, keepdims=True)
        idx = jnp.min(jnp.where(d == m, iota, BIG), axis=0, keepdims=True)
        ms.append(m)
        idxs.append(idx)
        if k < 2:
            d = jnp.where(iota == idx, jnp.inf, d)

    rs = [1.0 / (m + 1e-08) for m in ms]
    total = rs[0] + rs[1] + rs[2]
    wmat = (jnp.where(iota == idxs[0], rs[0], 0.0)
            + jnp.where(iota == idxs[1], rs[1], 0.0)
            + jnp.where(iota == idxs[2], rs[2], 0.0))

    interp = jnp.dot(p2_ref[0], wmat, preferred_element_type=jnp.float32)
    interp = interp * (1.0 / total)

    w0 = w0_ref[...]
    contrib = jnp.concatenate(
        [jnp.dot(p1_ref[0], w0, preferred_element_type=jnp.float32),
         jnp.dot(interp, w0, preferred_element_type=jnp.float32)], axis=0)

    @pl.when(t == 0)
    def _():
        acc_ref[...] = jnp.zeros_like(acc_ref)

    acc_ref[...] += contrib

    @pl.when(t == NT - 1)
    def _():
        h = acc_ref[...] + b0_ref[...]
        h = jnp.maximum(h * INV_BN * g0_ref[...] + beta0_ref[...], 0.0)
        h = jnp.dot(h, w1_ref[...], preferred_element_type=jnp.float32)
        h = h + b1_ref[...]
        h = jnp.maximum(h * INV_BN * g1_ref[...] + beta1_ref[...], 0.0)
        out_ref[0] = h


@jax.jit
def kernel(xyz1, xyz2, points1, points2, W0, b0, g0, beta0, W1, b1, g1, beta1):
    xyz2t = jnp.transpose(xyz2, (0, 2, 1))

    grid = (B, NT)
    out = pl.pallas_call(
        _body,
        grid=grid,
        in_specs=[
            pl.BlockSpec((1, 3, TN), lambda b, t: (b, 0, t)),
            pl.BlockSpec((1, S, 3), lambda b, t: (b, 0, 0)),
            pl.BlockSpec((1, D, TN), lambda b, t: (b, 0, t)),
            pl.BlockSpec((1, D, S), lambda b, t: (b, 0, 0)),
            pl.BlockSpec((TN, H0), lambda b, t: (t, 0)),
            pl.BlockSpec((1, H0), lambda b, t: (0, 0)),
            pl.BlockSpec((1, H0), lambda b, t: (0, 0)),
            pl.BlockSpec((1, H0), lambda b, t: (0, 0)),
            pl.BlockSpec((H0, H1), lambda b, t: (0, 0)),
            pl.BlockSpec((1, H1), lambda b, t: (0, 0)),
            pl.BlockSpec((1, H1), lambda b, t: (0, 0)),
            pl.BlockSpec((1, H1), lambda b, t: (0, 0)),
        ],
        out_specs=pl.BlockSpec((1, 2 * D, H1), lambda b, t: (b, 0, 0)),
        out_shape=jax.ShapeDtypeStruct((B, 2 * D, H1), jnp.float32),
        scratch_shapes=[pltpu.VMEM((2 * D, H0), jnp.float32)],
    )(xyz1, xyz2t, points1, points2, W0,
      b0.reshape(1, H0), g0.reshape(1, H0), beta0.reshape(1, H0),
      W1, b1.reshape(1, H1), g1.reshape(1, H1), beta1.reshape(1, H1))
    return out


# speedup vs baseline: 52.2223x; 1.0357x over previous
import functools

import jax
import jax.numpy as jnp
from jax.experimental import pallas as pl
from jax.experimental.pallas import tpu as pltpu

B, N, S, D = 8, 4096, 1024, 128
H0, H1 = 256, 128
TN = 4096
NT = N // TN
INV_BN = 1.0 / (1.0 + 1e-05) ** 0.5


def _body(x1_ref, x2t_ref, p1_ref, p2_ref, w0_ref, b0_ref, g0_ref,
          beta0_ref, w1_ref, b1_ref, g1_ref, beta1_ref, out_ref, acc_ref):
    t = pl.program_id(1)

    x1 = x1_ref[0]
    x2t = x2t_ref[0]

    cross = jnp.dot(x2t, -2.0 * x1, preferred_element_type=jnp.float32)
    x1sq = jnp.sum(x1 * x1, axis=0, keepdims=True)
    x2sq = jnp.sum(x2t * x2t, axis=1, keepdims=True)
    dist = (cross + x1sq) + x2sq

    iota = jax.lax.broadcasted_iota(jnp.int32, (S, TN), 0).astype(jnp.float32)
    BIG = float(S)

    d = dist
    ms, idxs = [], []
    for k in range(3):
        m = jnp.min(d, axis=0, keepdims=True)
        idx = jnp.min(jnp.where(d == m, iota, BIG), axis=0, keepdims=True)
        ms.append(m)
        idxs.append(idx)
        if k < 2:
            d = jnp.where(iota == idx, jnp.inf, d)

    rs = [1.0 / (m + 1e-08) for m in ms]
    total = rs[0] + rs[1] + rs[2]
    wmat = (jnp.where(iota == idxs[0], rs[0], 0.0)
            + jnp.where(iota == idxs[1], rs[1], 0.0)
            + jnp.where(iota == idxs[2], rs[2], 0.0))

    interp = jnp.dot(p2_ref[0], wmat, preferred_element_type=jnp.float32)
    interp = interp * (1.0 / total)

    w0 = w0_ref[...]
    contrib = jnp.concatenate(
        [jnp.dot(p1_ref[0], w0, preferred_element_type=jnp.float32),
         jnp.dot(interp, w0, preferred_element_type=jnp.float32)], axis=0)

    @pl.when(t == 0)
    def _():
        acc_ref[...] = jnp.zeros_like(acc_ref)

    acc_ref[...] += contrib

    @pl.when(t == NT - 1)
    def _():
        h = acc_ref[...] + b0_ref[...]
        h = jnp.maximum(h * INV_BN * g0_ref[...] + beta0_ref[...], 0.0)
        h = jnp.dot(h, w1_ref[...], preferred_element_type=jnp.float32)
        h = h + b1_ref[...]
        h = jnp.maximum(h * INV_BN * g1_ref[...] + beta1_ref[...], 0.0)
        out_ref[0] = h


@jax.jit
def kernel(xyz1, xyz2, points1, points2, W0, b0, g0, beta0, W1, b1, g1, beta1):
    xyz2t = jnp.transpose(xyz2, (0, 2, 1))

    grid = (B, NT)
    out = pl.pallas_call(
        _body,
        grid=grid,
        in_specs=[
            pl.BlockSpec((1, 3, TN), lambda b, t: (b, 0, t)),
            pl.BlockSpec((1, S, 3), lambda b, t: (b, 0, 0)),
            pl.BlockSpec((1, D, TN), lambda b, t: (b, 0, t)),
            pl.BlockSpec((1, D, S), lambda b, t: (b, 0, 0)),
            pl.BlockSpec((TN, H0), lambda b, t: (t, 0)),
            pl.BlockSpec((1, H0), lambda b, t: (0, 0)),
            pl.BlockSpec((1, H0), lambda b, t: (0, 0)),
            pl.BlockSpec((1, H0), lambda b, t: (0, 0)),
            pl.BlockSpec((H0, H1), lambda b, t: (0, 0)),
            pl.BlockSpec((1, H1), lambda b, t: (0, 0)),
            pl.BlockSpec((1, H1), lambda b, t: (0, 0)),
            pl.BlockSpec((1, H1), lambda b, t: (0, 0)),
        ],
        out_specs=pl.BlockSpec((1, 2 * D, H1), lambda b, t: (b, 0, 0)),
        out_shape=jax.ShapeDtypeStruct((B, 2 * D, H1), jnp.float32),
        scratch_shapes=[pltpu.VMEM((2 * D, H0), jnp.float32)],
    )(xyz1, xyz2t, points1, points2, W0,
      b0.reshape(1, H0), g0.reshape(1, H0), beta0.reshape(1, H0),
      W1, b1.reshape(1, H1), g1.reshape(1, H1), beta1.reshape(1, H1))
    return out
